# Initial kernel scaffold; baseline (speedup 1.0000x reference)
import jax
import jax.numpy as jnp
from jax.experimental import pallas as pl


def kernel(h, x, edge_index, a_ij, We1, be1, We2, be2, Wx1, bx1, Wx2, bx2, Wh1, bh1, Wh2, bh2, Winf, binf):
    raise NotImplementedError("write your pallas kernel here")


# speedup vs baseline: 1.3251x; 1.3251x over previous
import functools

import jax
import jax.numpy as jnp
from jax import lax
from jax.experimental import pallas as pl
from jax.experimental.pallas import tpu as pltpu
from jax.experimental.pallas import tpu_sc as plsc

N_NODES = 10000
E_EDGES = 320000
D = 128
NC, NS = 2, 16
NW = NC * NS
CH = 128
EPW = 10240
E_PAD = NW * EPW
XW = 8
NROW = 10240
ZR = NROW // NS


@functools.lru_cache(maxsize=None)
def _sc_kernels():
    mesh = plsc.VectorSubcoreMesh(core_axis_name="c", subcore_axis_name="s")

    @functools.partial(
        pl.kernel,
        mesh=mesh,
        out_type=(jax.ShapeDtypeStruct((E_PAD, D), jnp.float32),
                  jax.ShapeDtypeStruct((E_PAD, D), jnp.float32),
                  jax.ShapeDtypeStruct((E_PAD, D), jnp.float32),
                  jax.ShapeDtypeStruct((E_PAD, D), jnp.float32)),
        scratch_types=[
            pltpu.VMEM((CH,), jnp.int32),
            pltpu.VMEM((CH,), jnp.int32),
            pltpu.VMEM((CH, D), jnp.float32),
            pltpu.VMEM((CH, D), jnp.float32),
            pltpu.VMEM((CH, D), jnp.float32),
            pltpu.VMEM((CH, D), jnp.float32),
            pltpu.SemaphoreType.DMA,
            pltpu.SemaphoreType.DMA,
            pltpu.SemaphoreType.DMA,
            pltpu.SemaphoreType.DMA,
        ],
    )
    def _sc_gather(htbl_hbm, xtbl_hbm, i_hbm, j_hbm,
                   hi_hbm, hj_hbm, xi_hbm, xj_hbm,
                   iv, jv, bhi, bhj, bxi, bxj, s1, s2, s3, s4):
        wid = lax.axis_index("s") * NC + lax.axis_index("c")
        base = wid * EPW

        @pl.loop(0, EPW // CH)
        def _(c):
            off = base + c * CH
            pltpu.sync_copy(i_hbm.at[pl.ds(off, CH)], iv)
            pltpu.sync_copy(j_hbm.at[pl.ds(off, CH)], jv)
            pltpu.async_copy(htbl_hbm.at[iv], bhi, s1).wait()
            pltpu.async_copy(htbl_hbm.at[jv], bhj, s2).wait()
            pltpu.async_copy(xtbl_hbm.at[iv], bxi, s3).wait()
            pltpu.async_copy(xtbl_hbm.at[jv], bxj, s4).wait()
            pltpu.sync_copy(bhi, hi_hbm.at[pl.ds(off, CH)])
            pltpu.sync_copy(bhj, hj_hbm.at[pl.ds(off, CH)])
            pltpu.sync_copy(bxi, xi_hbm.at[pl.ds(off, CH)])
            pltpu.sync_copy(bxj, xj_hbm.at[pl.ds(off, CH)])

    @functools.partial(
        pl.kernel,
        mesh=mesh,
        out_type=(jax.ShapeDtypeStruct((NC, NROW, D), jnp.float32),
                  jax.ShapeDtypeStruct((NC, NROW, D), jnp.float32)),
        scratch_types=[
            pltpu.VMEM((CH,), jnp.int32),
            pltpu.VMEM((CH, D), jnp.float32),
            pltpu.VMEM_SHARED((NROW, D), jnp.float32),
        ],
    )
    def _sc_scatter(paym_hbm, payx_hbm, j_hbm, zm_hbm, outm_hbm, outx_hbm,
                    jv, bufm, accm):
        cid = lax.axis_index("c")
        sid = lax.axis_index("s")
        base = cid * (E_PAD // NC) + sid * EPW

        for pay_hbm, out_hbm in ((paym_hbm, outm_hbm), (payx_hbm, outx_hbm)):
            pltpu.sync_copy(zm_hbm, accm.at[pl.ds(sid * ZR, ZR)])
            plsc.subcore_barrier()

            @pl.loop(0, EPW // CH)
            def _(c):
                off = base + c * CH
                pltpu.sync_copy(j_hbm.at[pl.ds(off, CH)], jv)
                pltpu.sync_copy(pay_hbm.at[pl.ds(off, CH)], bufm)
                pltpu.sync_copy(bufm, accm.at[jv], add=True)

            plsc.subcore_barrier()
            pltpu.sync_copy(accm.at[pl.ds(sid * ZR, ZR)],
                            out_hbm.at[cid, pl.ds(sid * ZR, ZR)])
            plsc.subcore_barrier()

    return _sc_gather, _sc_scatter


def _edge_body(hi_ref, hj_ref, xi_ref, xj_ref, a_ref, we1_ref, be1_ref,
               we2_ref, be2_ref, wx1_ref, bx1_ref, wx2_ref, bx2_ref,
               winf_ref, binf_ref, outm_ref, outx_ref):
    diff = xi_ref[:, :XW] - xj_ref[:, :XW]
    dist = jnp.sqrt(jnp.sum(diff * diff, axis=1, keepdims=True) + 1e-12)

    extras = jnp.concatenate([dist, a_ref[...]], axis=1)
    hhda = jnp.concatenate(
        [hi_ref[...], hj_ref[...], extras],
        axis=1).astype(jnp.bfloat16)

    e1 = jnp.dot(hhda, we1_ref[...],
                 preferred_element_type=jnp.float32) + be1_ref[...]
    e1 = e1 * jax.nn.sigmoid(e1)
    m1 = jnp.dot(e1.astype(jnp.bfloat16), we2_ref[...],
                 preferred_element_type=jnp.float32) + be2_ref[...]
    m_ij = m1 * jax.nn.sigmoid(m1)

    x1 = jnp.dot(hhda, wx1_ref[...],
                 preferred_element_type=jnp.float32) + bx1_ref[...]
    x1 = x1 * jax.nn.sigmoid(x1)
    w_x = jnp.sum(x1 * wx2_ref[...], axis=1, keepdims=True) + bx2_ref[...]

    gate = jax.nn.sigmoid(
        jnp.sum(m_ij * winf_ref[...], axis=1, keepdims=True) + binf_ref[...])

    cnt_col = (lax.broadcasted_iota(jnp.int32, (1, D), 1) == 3)
    diff_w = jnp.pad(diff * w_x, ((0, 0), (0, D - XW)))
    outm_ref[...] = gate * m_ij
    outx_ref[...] = diff_w + cnt_col.astype(jnp.float32)


BE = 2048


def _edge_call(hi, hj, xi, xj, a7, we1p, be1r, we2b, be2r, wx1p, bx1r, wx2r,
               bx2s, winfr, binfs):
    full = lambda shape: pl.BlockSpec(shape, lambda i: (0, 0))
    row = lambda width: pl.BlockSpec((BE, width), lambda i: (i, 0))
    return pl.pallas_call(
        _edge_body,
        grid=(E_PAD // BE,),
        in_specs=[
            row(D), row(D), row(D), row(D), row(7),
            full((264, D)), full((1, D)), full((D, D)), full((1, D)),
            full((264, D)), full((1, D)), full((1, D)), full((1, 1)),
            full((1, D)), full((1, 1)),
        ],
        out_specs=[row(D), row(D)],
        out_shape=[jax.ShapeDtypeStruct((E_PAD, D), jnp.float32),
                   jax.ShapeDtypeStruct((E_PAD, D), jnp.float32)],
    )(hi, hj, xi, xj, a7, we1p, be1r, we2b, be2r, wx1p, bx1r, wx2r, bx2s,
      winfr, binfs)


def _node_body(h_ref, xs_ref, m0_ref, m1_ref, p0_ref, p1_ref, wh1_ref,
               bh1_ref, wh2_ref, bh2_ref, hout_ref, xout_ref):
    dwc = p0_ref[:, :XW] + p1_ref[:, :XW]
    cnt_col = (lax.broadcasted_iota(jnp.int32, (1, XW), 1) == 3)
    cnt = jnp.sum(dwc * cnt_col.astype(jnp.float32), axis=1, keepdims=True)
    xout_ref[...] = xs_ref[...] + dwc / jnp.maximum(cnt, 1.0)

    h = h_ref[...]
    m = m0_ref[...] + m1_ref[...]
    hcat = jnp.concatenate([h, m], axis=1).astype(jnp.bfloat16)
    t = jnp.dot(hcat, wh1_ref[...],
                preferred_element_type=jnp.float32) + bh1_ref[...]
    t = t * jax.nn.sigmoid(t)
    hout_ref[...] = h + jnp.dot(t.astype(jnp.bfloat16), wh2_ref[...],
                                preferred_element_type=jnp.float32) + bh2_ref[...]


BN = 1000


def _node_call(h, xs, m0, m1, p0, p1, wh1b, bh1r, wh2b, bh2r):
    full = lambda shape: pl.BlockSpec(shape, lambda i: (0, 0))
    row = lambda width: pl.BlockSpec((BN, width), lambda i: (i, 0))
    return pl.pallas_call(
        _node_body,
        grid=(N_NODES // BN,),
        in_specs=[
            row(D), row(XW), row(D), row(D), row(D), row(D),
            full((2 * D, D)), full((1, D)), full((D, D)), full((1, D)),
        ],
        out_specs=[row(D), row(XW)],
        out_shape=[jax.ShapeDtypeStruct((N_NODES, D), jnp.float32),
                   jax.ShapeDtypeStruct((N_NODES, XW), jnp.float32)],
    )(h, xs, m0, m1, p0, p1, wh1b, bh1r, wh2b, bh2r)


def kernel(h, x, edge_index, a_ij, We1, be1, We2, be2, Wx1, bx1, Wx2, bx2,
           Wh1, bh1, Wh2, bh2, Winf, binf):
    f32 = jnp.float32
    bf16 = jnp.bfloat16

    htbl = h
    xtbl = jnp.pad(x, ((0, 0), (0, D - 3)))

    pad_e = E_PAD - E_EDGES
    i_idx = jnp.pad(edge_index[0], (0, pad_e))
    jg_idx = jnp.pad(edge_index[1], (0, pad_e))
    js_idx = jnp.pad(edge_index[1], (0, pad_e),
                     constant_values=N_NODES)
    a7 = jnp.pad(a_ij, ((0, pad_e), (0, 3)))

    pack = lambda W: jnp.concatenate([W, jnp.zeros((3, D), f32)], axis=0)
    we1p = pack(We1).astype(bf16)
    wx1p = pack(Wx1).astype(bf16)
    we2b = We2.astype(bf16)
    wh1b = Wh1.astype(bf16)
    wh2b = Wh2.astype(bf16)
    r = lambda v: v.reshape(1, -1)

    sc_gather, sc_scatter = _sc_kernels()
    hi, hj, xi, xj = sc_gather(htbl, xtbl, i_idx, jg_idx)
    paym, payx = _edge_call(hi, hj, xi, xj, a7, we1p, r(be1), we2b, r(be2),
                            wx1p, r(bx1), r(Wx2), r(bx2), r(Winf), r(binf))
    zm = jnp.zeros((ZR, D), f32)
    pm, px = sc_scatter(paym, payx, js_idx, zm)
    xs = jnp.pad(x, ((0, 0), (0, XW - 3)))
    h_new, x8 = _node_call(h, xs, pm[0, :N_NODES], pm[1, :N_NODES],
                           px[0, :N_NODES], px[1, :N_NODES],
                           wh1b, r(bh1), wh2b, r(bh2))
    return (h_new, x8[:, :3])


# speedup vs baseline: 2.2905x; 1.7285x over previous
import functools

import jax
import jax.numpy as jnp
from jax import lax
from jax.experimental import pallas as pl
from jax.experimental.pallas import tpu as pltpu
from jax.experimental.pallas import tpu_sc as plsc

N_NODES = 10000
E_EDGES = 320000
D = 128
HD = D // 2
NC, NS = 2, 16
NW = NC * NS
CH = 128
EPW = 10240
E_PAD = NW * EPW
XW = 8
NROW = 10240
ZR = NROW // NS


@functools.lru_cache(maxsize=None)
def _sc_kernels():
    mesh = plsc.VectorSubcoreMesh(core_axis_name="c", subcore_axis_name="s")

    @functools.partial(
        pl.kernel,
        mesh=mesh,
        out_type=(jax.ShapeDtypeStruct((E_PAD, D), jnp.int32),
                  jax.ShapeDtypeStruct((E_PAD, D), jnp.int32)),
        scratch_types=[
            pltpu.VMEM((CH,), jnp.int32),
            pltpu.VMEM((CH,), jnp.int32),
            pltpu.VMEM((CH, D), jnp.int32),
            pltpu.VMEM((CH, D), jnp.int32),
            pltpu.SemaphoreType.DMA,
            pltpu.SemaphoreType.DMA,
        ],
    )
    def _sc_gather(tbl_hbm, i_hbm, j_hbm, ti_hbm, tj_hbm,
                   iv, jv, bi, bj, s1, s2):
        wid = lax.axis_index("s") * NC + lax.axis_index("c")
        base = wid * EPW

        @pl.loop(0, EPW // CH)
        def _(c):
            off = base + c * CH
            pltpu.sync_copy(i_hbm.at[pl.ds(off, CH)], iv)
            pltpu.sync_copy(j_hbm.at[pl.ds(off, CH)], jv)
            cp1 = pltpu.async_copy(tbl_hbm.at[iv], bi, s1)
            cp2 = pltpu.async_copy(tbl_hbm.at[jv], bj, s2)
            cp1.wait()
            cp2.wait()
            pltpu.sync_copy(bi, ti_hbm.at[pl.ds(off, CH)])
            pltpu.sync_copy(bj, tj_hbm.at[pl.ds(off, CH)])

    @functools.partial(
        pl.kernel,
        mesh=mesh,
        out_type=(jax.ShapeDtypeStruct((NC, NROW, D), jnp.float32),
                  jax.ShapeDtypeStruct((NC, NROW, D), jnp.float32)),
        scratch_types=[
            pltpu.VMEM((CH,), jnp.int32),
            pltpu.VMEM((CH, D), jnp.float32),
            pltpu.VMEM_SHARED((NROW, D), jnp.float32),
        ],
    )
    def _sc_scatter(paym_hbm, payx_hbm, j_hbm, zm_hbm, outm_hbm, outx_hbm,
                    jv, bufm, accm):
        cid = lax.axis_index("c")
        sid = lax.axis_index("s")
        base = cid * (E_PAD // NC) + sid * EPW

        for pay_hbm, out_hbm in ((paym_hbm, outm_hbm), (payx_hbm, outx_hbm)):
            pltpu.sync_copy(zm_hbm, accm.at[pl.ds(sid * ZR, ZR)])
            plsc.subcore_barrier()

            @pl.loop(0, EPW // CH)
            def _(c):
                off = base + c * CH
                pltpu.sync_copy(j_hbm.at[pl.ds(off, CH)], jv)
                pltpu.sync_copy(pay_hbm.at[pl.ds(off, CH)], bufm)
                pltpu.sync_copy(bufm, accm.at[jv], add=True)

            plsc.subcore_barrier()
            pltpu.sync_copy(accm.at[pl.ds(sid * ZR, ZR)],
                            out_hbm.at[cid, pl.ds(sid * ZR, ZR)])
            plsc.subcore_barrier()

    return _sc_gather, _sc_scatter


def _unpack(t):
    hw = t[:, :HD]
    ha = lax.bitcast_convert_type(hw << 16, jnp.float32)
    hb = lax.bitcast_convert_type(hw & jnp.int32(-65536),
                                  jnp.float32)
    h = jnp.concatenate([ha, hb], axis=1).astype(jnp.bfloat16)
    x = lax.bitcast_convert_type(t[:, HD:HD + XW], jnp.float32)
    return h, x


def _edge_body(ti_ref, tj_ref, a_ref, we1_ref, be1_ref, we2_ref, be2_ref,
               wx1_ref, bx1_ref, wx2_ref, bx2_ref, winf_ref, binf_ref,
               outm_ref, outx_ref):
    hi, xi = _unpack(ti_ref[...])
    hj, xj = _unpack(tj_ref[...])
    diff = xi - xj
    dist = jnp.sqrt(jnp.sum(diff * diff, axis=1, keepdims=True) + 1e-12)

    extras = jnp.concatenate([dist, a_ref[...]], axis=1)
    hhda = jnp.concatenate(
        [hi, hj, extras.astype(jnp.bfloat16)], axis=1)

    e1 = jnp.dot(hhda, we1_ref[...],
                 preferred_element_type=jnp.float32) + be1_ref[...]
    e1 = e1 * jax.nn.sigmoid(e1)
    m1 = jnp.dot(e1.astype(jnp.bfloat16), we2_ref[...],
                 preferred_element_type=jnp.float32) + be2_ref[...]
    m_ij = m1 * jax.nn.sigmoid(m1)

    x1 = jnp.dot(hhda, wx1_ref[...],
                 preferred_element_type=jnp.float32) + bx1_ref[...]
    x1 = x1 * jax.nn.sigmoid(x1)
    w_x = jnp.sum(x1 * wx2_ref[...], axis=1, keepdims=True) + bx2_ref[...]

    gate = jax.nn.sigmoid(
        jnp.sum(m_ij * winf_ref[...], axis=1, keepdims=True) + binf_ref[...])

    cnt_col = (lax.broadcasted_iota(jnp.int32, (1, D), 1) == 3)
    diff_w = jnp.pad(diff * w_x, ((0, 0), (0, D - XW)))
    outm_ref[...] = gate * m_ij
    outx_ref[...] = diff_w + cnt_col.astype(jnp.float32)


BE = 2048


def _edge_call(ti, tj, a7, we1p, be1r, we2b, be2r, wx1p, bx1r, wx2r, bx2s,
               winfr, binfs):
    full = lambda shape: pl.BlockSpec(shape, lambda i: (0, 0))
    row = lambda width: pl.BlockSpec((BE, width), lambda i: (i, 0))
    return pl.pallas_call(
        _edge_body,
        grid=(E_PAD // BE,),
        in_specs=[
            row(D), row(D), row(7),
            full((264, D)), full((1, D)), full((D, D)), full((1, D)),
            full((264, D)), full((1, D)), full((1, D)), full((1, 1)),
            full((1, D)), full((1, 1)),
        ],
        out_specs=[row(D), row(D)],
        out_shape=[jax.ShapeDtypeStruct((E_PAD, D), jnp.float32),
                   jax.ShapeDtypeStruct((E_PAD, D), jnp.float32)],
    )(ti, tj, a7, we1p, be1r, we2b, be2r, wx1p, bx1r, wx2r, bx2s, winfr,
      binfs)


def _node_body(h_ref, xs_ref, m0_ref, m1_ref, p0_ref, p1_ref, wh1_ref,
               bh1_ref, wh2_ref, bh2_ref, hout_ref, xout_ref):
    dwc = p0_ref[:, :XW] + p1_ref[:, :XW]
    cnt_col = (lax.broadcasted_iota(jnp.int32, (1, XW), 1) == 3)
    cnt = jnp.sum(dwc * cnt_col.astype(jnp.float32), axis=1, keepdims=True)
    xout_ref[...] = xs_ref[...] + dwc / jnp.maximum(cnt, 1.0)

    h = h_ref[...]
    m = m0_ref[...] + m1_ref[...]
    hcat = jnp.concatenate([h, m], axis=1).astype(jnp.bfloat16)
    t = jnp.dot(hcat, wh1_ref[...],
                preferred_element_type=jnp.float32) + bh1_ref[...]
    t = t * jax.nn.sigmoid(t)
    hout_ref[...] = h + jnp.dot(t.astype(jnp.bfloat16), wh2_ref[...],
                                preferred_element_type=jnp.float32) + bh2_ref[...]


BN = 1000


def _node_call(h, xs, m0, m1, p0, p1, wh1b, bh1r, wh2b, bh2r):
    full = lambda shape: pl.BlockSpec(shape, lambda i: (0, 0))
    row = lambda width: pl.BlockSpec((BN, width), lambda i: (i, 0))
    return pl.pallas_call(
        _node_body,
        grid=(N_NODES // BN,),
        in_specs=[
            row(D), row(XW), row(D), row(D), row(D), row(D),
            full((2 * D, D)), full((1, D)), full((D, D)), full((1, D)),
        ],
        out_specs=[row(D), row(XW)],
        out_shape=[jax.ShapeDtypeStruct((N_NODES, D), jnp.float32),
                   jax.ShapeDtypeStruct((N_NODES, XW), jnp.float32)],
    )(h, xs, m0, m1, p0, p1, wh1b, bh1r, wh2b, bh2r)


def _pack_table(h, x):
    hb = h.astype(jnp.bfloat16)
    lo = lax.bitcast_convert_type(hb[:, :HD], jnp.uint16).astype(jnp.uint32)
    hi = lax.bitcast_convert_type(hb[:, HD:], jnp.uint16).astype(jnp.uint32)
    hw = lax.bitcast_convert_type(lo | (hi << 16), jnp.int32)
    xw = lax.bitcast_convert_type(
        jnp.pad(x, ((0, 0), (0, HD - 3))), jnp.int32)
    return jnp.concatenate([hw, xw[:, :D - HD]], axis=1)


def kernel(h, x, edge_index, a_ij, We1, be1, We2, be2, Wx1, bx1, Wx2, bx2,
           Wh1, bh1, Wh2, bh2, Winf, binf):
    f32 = jnp.float32
    bf16 = jnp.bfloat16

    tbl = _pack_table(h, x)

    pad_e = E_PAD - E_EDGES
    i_idx = jnp.pad(edge_index[0], (0, pad_e))
    jg_idx = jnp.pad(edge_index[1], (0, pad_e))
    js_idx = jnp.pad(edge_index[1], (0, pad_e),
                     constant_values=N_NODES)
    a7 = jnp.pad(a_ij, ((0, pad_e), (0, 3)))

    pack = lambda W: jnp.concatenate([W, jnp.zeros((3, D), f32)], axis=0)
    we1p = pack(We1).astype(bf16)
    wx1p = pack(Wx1).astype(bf16)
    we2b = We2.astype(bf16)
    wh1b = Wh1.astype(bf16)
    wh2b = Wh2.astype(bf16)
    r = lambda v: v.reshape(1, -1)

    sc_gather, sc_scatter = _sc_kernels()
    ti, tj = sc_gather(tbl, i_idx, jg_idx)
    paym, payx = _edge_call(ti, tj, a7, we1p, r(be1), we2b, r(be2), wx1p,
                            r(bx1), r(Wx2), r(bx2), r(Winf), r(binf))
    zm = jnp.zeros((ZR, D), f32)
    pm, px = sc_scatter(paym, payx, js_idx, zm)
    xs = jnp.pad(x, ((0, 0), (0, XW - 3)))
    h_new, x8 = _node_call(h, xs, pm[0, :N_NODES], pm[1, :N_NODES],
                           px[0, :N_NODES], px[1, :N_NODES],
                           wh1b, r(bh1), wh2b, r(bh2))
    return (h_new, x8[:, :3])


# speedup vs baseline: 2.4623x; 1.0750x over previous
import functools

import jax
import jax.numpy as jnp
from jax import lax
from jax.experimental import pallas as pl
from jax.experimental.pallas import tpu as pltpu
from jax.experimental.pallas import tpu_sc as plsc

N_NODES = 10000
E_EDGES = 320000
D = 128
HD = D // 2
NC, NS = 2, 16
NW = NC * NS
CH = 128
EPW = 10240
E_PAD = NW * EPW
XW = 8
NROW = 10240
ZR = NROW // NS


@functools.lru_cache(maxsize=None)
def _sc_kernels():
    mesh = plsc.VectorSubcoreMesh(core_axis_name="c", subcore_axis_name="s")

    @functools.partial(
        pl.kernel,
        mesh=mesh,
        out_type=(jax.ShapeDtypeStruct((E_PAD, D), jnp.int32),
                  jax.ShapeDtypeStruct((E_PAD, D), jnp.int32)),
        scratch_types=[
            pltpu.VMEM((EPW,), jnp.int32),
            pltpu.VMEM((EPW,), jnp.int32),
            pltpu.VMEM((CH, D), jnp.int32),
            pltpu.VMEM((CH, D), jnp.int32),
            pltpu.VMEM((CH, D), jnp.int32),
            pltpu.VMEM((CH, D), jnp.int32),
            pltpu.SemaphoreType.DMA,
            pltpu.SemaphoreType.DMA,
            pltpu.SemaphoreType.DMA,
            pltpu.SemaphoreType.DMA,
        ],
    )
    def _sc_gather(tbl_hbm, i_hbm, j_hbm, ti_hbm, tj_hbm,
                   iva, jva, bi0, bj0, bi1, bj1, sg0, sg1, sw0, sw1):
        wid = lax.axis_index("s") * NC + lax.axis_index("c")
        base = wid * EPW
        G = EPW // CH // 2

        pltpu.sync_copy(i_hbm.at[pl.ds(base, EPW)], iva)
        pltpu.sync_copy(j_hbm.at[pl.ds(base, EPW)], jva)

        ixi = lambda c: tbl_hbm.at[iva.at[pl.ds(c * CH, CH)]]
        ixj = lambda c: tbl_hbm.at[jva.at[pl.ds(c * CH, CH)]]
        out_i = lambda c: ti_hbm.at[pl.ds(base + c * CH, CH)]
        out_j = lambda c: tj_hbm.at[pl.ds(base + c * CH, CH)]

        pltpu.async_copy(ixi(0), bi0, sg0)
        pltpu.async_copy(ixj(0), bj0, sg0)

        @pl.loop(0, G)
        def _(g):
            c0 = 2 * g
            c1 = c0 + 1
            @pl.when(g > 0)
            def _():
                pltpu.make_async_copy(bi1, out_i(c1), sw1).wait()
                pltpu.make_async_copy(bj1, out_j(c1), sw1).wait()
            pltpu.async_copy(ixi(c1), bi1, sg1)
            pltpu.async_copy(ixj(c1), bj1, sg1)
            pltpu.make_async_copy(ixi(c0), bi0, sg0).wait()
            pltpu.make_async_copy(ixj(c0), bj0, sg0).wait()
            pltpu.async_copy(bi0, out_i(c0), sw0)
            pltpu.async_copy(bj0, out_j(c0), sw0)
            pltpu.make_async_copy(bi0, out_i(c0), sw0).wait()
            pltpu.make_async_copy(bj0, out_j(c0), sw0).wait()

            @pl.when(g < G - 1)
            def _():
                pltpu.async_copy(ixi(c0 + 2), bi0, sg0)
                pltpu.async_copy(ixj(c0 + 2), bj0, sg0)
            pltpu.make_async_copy(ixi(c1), bi1, sg1).wait()
            pltpu.make_async_copy(ixj(c1), bj1, sg1).wait()
            pltpu.async_copy(bi1, out_i(c1), sw1)
            pltpu.async_copy(bj1, out_j(c1), sw1)

        pltpu.make_async_copy(bi1, out_i(1), sw1).wait()
        pltpu.make_async_copy(bj1, out_j(1), sw1).wait()

    @functools.partial(
        pl.kernel,
        mesh=mesh,
        out_type=(jax.ShapeDtypeStruct((NC, NROW, D), jnp.float32),
                  jax.ShapeDtypeStruct((NC, NROW, D), jnp.float32)),
        scratch_types=[
            pltpu.VMEM((CH,), jnp.int32),
            pltpu.VMEM((CH, D), jnp.float32),
            pltpu.VMEM_SHARED((NROW, D), jnp.float32),
        ],
    )
    def _sc_scatter(paym_hbm, payx_hbm, j_hbm, zm_hbm, outm_hbm, outx_hbm,
                    jv, bufm, accm):
        cid = lax.axis_index("c")
        sid = lax.axis_index("s")
        base = cid * (E_PAD // NC) + sid * EPW

        for pay_hbm, out_hbm in ((paym_hbm, outm_hbm), (payx_hbm, outx_hbm)):
            pltpu.sync_copy(zm_hbm, accm.at[pl.ds(sid * ZR, ZR)])
            plsc.subcore_barrier()

            @pl.loop(0, EPW // CH)
            def _(c):
                off = base + c * CH
                pltpu.sync_copy(j_hbm.at[pl.ds(off, CH)], jv)
                pltpu.sync_copy(pay_hbm.at[pl.ds(off, CH)], bufm)
                pltpu.sync_copy(bufm, accm.at[jv], add=True)

            plsc.subcore_barrier()
            pltpu.sync_copy(accm.at[pl.ds(sid * ZR, ZR)],
                            out_hbm.at[cid, pl.ds(sid * ZR, ZR)])
            plsc.subcore_barrier()

    return _sc_gather, _sc_scatter


def _unpack(t):
    hw = t[:, :HD]
    ha = lax.bitcast_convert_type(hw << 16, jnp.float32)
    hb = lax.bitcast_convert_type(hw & jnp.int32(-65536),
                                  jnp.float32)
    h = jnp.concatenate([ha, hb], axis=1).astype(jnp.bfloat16)
    x = lax.bitcast_convert_type(t[:, HD:HD + XW], jnp.float32)
    return h, x


def _edge_body(ti_ref, tj_ref, a_ref, we1_ref, be1_ref, we2_ref, be2_ref,
               wx1_ref, bx1_ref, wx2_ref, bx2_ref, winf_ref, binf_ref,
               outm_ref, outx_ref):
    hi, xi = _unpack(ti_ref[...])
    hj, xj = _unpack(tj_ref[...])
    diff = xi - xj
    dist = jnp.sqrt(jnp.sum(diff * diff, axis=1, keepdims=True) + 1e-12)

    extras = jnp.concatenate([dist, a_ref[...]], axis=1)
    hhda = jnp.concatenate(
        [hi, hj, extras.astype(jnp.bfloat16)], axis=1)

    e1 = jnp.dot(hhda, we1_ref[...],
                 preferred_element_type=jnp.float32) + be1_ref[...]
    e1 = e1 * jax.nn.sigmoid(e1)
    m1 = jnp.dot(e1.astype(jnp.bfloat16), we2_ref[...],
                 preferred_element_type=jnp.float32) + be2_ref[...]
    m_ij = m1 * jax.nn.sigmoid(m1)

    x1 = jnp.dot(hhda, wx1_ref[...],
                 preferred_element_type=jnp.float32) + bx1_ref[...]
    x1 = x1 * jax.nn.sigmoid(x1)
    w_x = jnp.sum(x1 * wx2_ref[...], axis=1, keepdims=True) + bx2_ref[...]

    gate = jax.nn.sigmoid(
        jnp.sum(m_ij * winf_ref[...], axis=1, keepdims=True) + binf_ref[...])

    cnt_col = (lax.broadcasted_iota(jnp.int32, (1, D), 1) == 3)
    diff_w = jnp.pad(diff * w_x, ((0, 0), (0, D - XW)))
    outm_ref[...] = gate * m_ij
    outx_ref[...] = diff_w + cnt_col.astype(jnp.float32)


BE = 2048


def _edge_call(ti, tj, a7, we1p, be1r, we2b, be2r, wx1p, bx1r, wx2r, bx2s,
               winfr, binfs):
    full = lambda shape: pl.BlockSpec(shape, lambda i: (0, 0))
    row = lambda width: pl.BlockSpec((BE, width), lambda i: (i, 0))
    return pl.pallas_call(
        _edge_body,
        grid=(E_PAD // BE,),
        in_specs=[
            row(D), row(D), row(7),
            full((264, D)), full((1, D)), full((D, D)), full((1, D)),
            full((264, D)), full((1, D)), full((1, D)), full((1, 1)),
            full((1, D)), full((1, 1)),
        ],
        out_specs=[row(D), row(D)],
        out_shape=[jax.ShapeDtypeStruct((E_PAD, D), jnp.float32),
                   jax.ShapeDtypeStruct((E_PAD, D), jnp.float32)],
    )(ti, tj, a7, we1p, be1r, we2b, be2r, wx1p, bx1r, wx2r, bx2s, winfr,
      binfs)


def _node_body(h_ref, xs_ref, m0_ref, m1_ref, p0_ref, p1_ref, wh1_ref,
               bh1_ref, wh2_ref, bh2_ref, hout_ref, xout_ref):
    dwc = p0_ref[:, :XW] + p1_ref[:, :XW]
    cnt_col = (lax.broadcasted_iota(jnp.int32, (1, XW), 1) == 3)
    cnt = jnp.sum(dwc * cnt_col.astype(jnp.float32), axis=1, keepdims=True)
    xout_ref[...] = xs_ref[...] + dwc / jnp.maximum(cnt, 1.0)

    h = h_ref[...]
    m = m0_ref[...] + m1_ref[...]
    hcat = jnp.concatenate([h, m], axis=1).astype(jnp.bfloat16)
    t = jnp.dot(hcat, wh1_ref[...],
                preferred_element_type=jnp.float32) + bh1_ref[...]
    t = t * jax.nn.sigmoid(t)
    hout_ref[...] = h + jnp.dot(t.astype(jnp.bfloat16), wh2_ref[...],
                                preferred_element_type=jnp.float32) + bh2_ref[...]


BN = 1000


def _node_call(h, xs, m0, m1, p0, p1, wh1b, bh1r, wh2b, bh2r):
    full = lambda shape: pl.BlockSpec(shape, lambda i: (0, 0))
    row = lambda width: pl.BlockSpec((BN, width), lambda i: (i, 0))
    return pl.pallas_call(
        _node_body,
        grid=(N_NODES // BN,),
        in_specs=[
            row(D), row(XW), row(D), row(D), row(D), row(D),
            full((2 * D, D)), full((1, D)), full((D, D)), full((1, D)),
        ],
        out_specs=[row(D), row(XW)],
        out_shape=[jax.ShapeDtypeStruct((N_NODES, D), jnp.float32),
                   jax.ShapeDtypeStruct((N_NODES, XW), jnp.float32)],
    )(h, xs, m0, m1, p0, p1, wh1b, bh1r, wh2b, bh2r)


def _pack_table(h, x):
    hb = h.astype(jnp.bfloat16)
    lo = lax.bitcast_convert_type(hb[:, :HD], jnp.uint16).astype(jnp.uint32)
    hi = lax.bitcast_convert_type(hb[:, HD:], jnp.uint16).astype(jnp.uint32)
    hw = lax.bitcast_convert_type(lo | (hi << 16), jnp.int32)
    xw = lax.bitcast_convert_type(
        jnp.pad(x, ((0, 0), (0, HD - 3))), jnp.int32)
    return jnp.concatenate([hw, xw[:, :D - HD]], axis=1)


def kernel(h, x, edge_index, a_ij, We1, be1, We2, be2, Wx1, bx1, Wx2, bx2,
           Wh1, bh1, Wh2, bh2, Winf, binf):
    f32 = jnp.float32
    bf16 = jnp.bfloat16

    tbl = _pack_table(h, x)

    pad_e = E_PAD - E_EDGES
    i_idx = jnp.pad(edge_index[0], (0, pad_e))
    jg_idx = jnp.pad(edge_index[1], (0, pad_e))
    js_idx = jnp.pad(edge_index[1], (0, pad_e),
                     constant_values=N_NODES)
    a7 = jnp.pad(a_ij, ((0, pad_e), (0, 3)))

    pack = lambda W: jnp.concatenate([W, jnp.zeros((3, D), f32)], axis=0)
    we1p = pack(We1).astype(bf16)
    wx1p = pack(Wx1).astype(bf16)
    we2b = We2.astype(bf16)
    wh1b = Wh1.astype(bf16)
    wh2b = Wh2.astype(bf16)
    r = lambda v: v.reshape(1, -1)

    sc_gather, sc_scatter = _sc_kernels()
    ti, tj = sc_gather(tbl, i_idx, jg_idx)
    paym, payx = _edge_call(ti, tj, a7, we1p, r(be1), we2b, r(be2), wx1p,
                            r(bx1), r(Wx2), r(bx2), r(Winf), r(binf))
    zm = jnp.zeros((ZR, D), f32)
    pm, px = sc_scatter(paym, payx, js_idx, zm)
    xs = jnp.pad(x, ((0, 0), (0, XW - 3)))
    h_new, x8 = _node_call(h, xs, pm[0, :N_NODES], pm[1, :N_NODES],
                           px[0, :N_NODES], px[1, :N_NODES],
                           wh1b, r(bh1), wh2b, r(bh2))
    return (h_new, x8[:, :3])


# speedup vs baseline: 3.4075x; 1.3839x over previous
import functools

import jax
import jax.numpy as jnp
from jax import lax
from jax.experimental import pallas as pl
from jax.experimental.pallas import tpu as pltpu
from jax.experimental.pallas import tpu_sc as plsc

N_NODES = 10000
E_EDGES = 320000
D = 128
HD = D // 2
NC, NS = 2, 16
NW = NC * NS
CH = 128
EPW = 10240
E_PAD = NW * EPW
XW = 8
NROW = 10240
ZR = NROW // NS


@functools.lru_cache(maxsize=None)
def _sc_kernels():
    mesh = plsc.VectorSubcoreMesh(core_axis_name="c", subcore_axis_name="s")

    @functools.partial(
        pl.kernel,
        mesh=mesh,
        out_type=(jax.ShapeDtypeStruct((E_PAD, D), jnp.int32),
                  jax.ShapeDtypeStruct((E_PAD, D), jnp.int32)),
        scratch_types=[
            pltpu.VMEM((EPW,), jnp.int32),
            pltpu.VMEM((EPW,), jnp.int32),
            pltpu.VMEM((CH, D), jnp.int32),
            pltpu.VMEM((CH, D), jnp.int32),
            pltpu.VMEM((CH, D), jnp.int32),
            pltpu.VMEM((CH, D), jnp.int32),
            pltpu.SemaphoreType.DMA,
            pltpu.SemaphoreType.DMA,
            pltpu.SemaphoreType.DMA,
            pltpu.SemaphoreType.DMA,
        ],
    )
    def _sc_gather(tbl_hbm, i_hbm, j_hbm, ti_hbm, tj_hbm,
                   iva, jva, bi0, bj0, bi1, bj1, sg0, sg1, sw0, sw1):
        base = (lax.axis_index("c") * (E_PAD // NC)
                + lax.axis_index("s") * EPW)
        G = EPW // CH // 2

        pltpu.sync_copy(i_hbm.at[pl.ds(base, EPW)], iva)
        pltpu.sync_copy(j_hbm.at[pl.ds(base, EPW)], jva)

        ixi = lambda c: tbl_hbm.at[iva.at[pl.ds(c * CH, CH)]]
        ixj = lambda c: tbl_hbm.at[jva.at[pl.ds(c * CH, CH)]]
        out_i = lambda c: ti_hbm.at[pl.ds(base + c * CH, CH)]
        out_j = lambda c: tj_hbm.at[pl.ds(base + c * CH, CH)]

        pltpu.async_copy(ixi(0), bi0, sg0)
        pltpu.async_copy(ixj(0), bj0, sg0)

        @pl.loop(0, G)
        def _(g):
            c0 = 2 * g
            c1 = c0 + 1
            @pl.when(g > 0)
            def _():
                pltpu.make_async_copy(bi1, out_i(c1), sw1).wait()
                pltpu.make_async_copy(bj1, out_j(c1), sw1).wait()
            pltpu.async_copy(ixi(c1), bi1, sg1)
            pltpu.async_copy(ixj(c1), bj1, sg1)
            pltpu.make_async_copy(ixi(c0), bi0, sg0).wait()
            pltpu.make_async_copy(ixj(c0), bj0, sg0).wait()
            pltpu.async_copy(bi0, out_i(c0), sw0)
            pltpu.async_copy(bj0, out_j(c0), sw0)
            pltpu.make_async_copy(bi0, out_i(c0), sw0).wait()
            pltpu.make_async_copy(bj0, out_j(c0), sw0).wait()

            @pl.when(g < G - 1)
            def _():
                pltpu.async_copy(ixi(c0 + 2), bi0, sg0)
                pltpu.async_copy(ixj(c0 + 2), bj0, sg0)
            pltpu.make_async_copy(ixi(c1), bi1, sg1).wait()
            pltpu.make_async_copy(ixj(c1), bj1, sg1).wait()
            pltpu.async_copy(bi1, out_i(c1), sw1)
            pltpu.async_copy(bj1, out_j(c1), sw1)

        pltpu.make_async_copy(bi1, out_i(1), sw1).wait()
        pltpu.make_async_copy(bj1, out_j(1), sw1).wait()

    @functools.partial(
        pl.kernel,
        mesh=mesh,
        out_type=(jax.ShapeDtypeStruct((NC, NROW, D), jnp.float32),
                  jax.ShapeDtypeStruct((NC, NROW, D), jnp.float32)),
        scratch_types=[
            pltpu.VMEM((CH,), jnp.int32),
            pltpu.VMEM((CH, D), jnp.float32),
            pltpu.VMEM_SHARED((NROW, D), jnp.float32),
        ],
    )
    def _sc_scatter(paym_hbm, payx_hbm, j_hbm, zm_hbm, outm_hbm, outx_hbm,
                    jv, bufm, accm):
        cid = lax.axis_index("c")
        sid = lax.axis_index("s")
        base = cid * (E_PAD // NC) + sid * EPW

        for pay_hbm, out_hbm in ((paym_hbm, outm_hbm), (payx_hbm, outx_hbm)):
            pltpu.sync_copy(zm_hbm, accm.at[pl.ds(sid * ZR, ZR)])
            plsc.subcore_barrier()

            @pl.loop(0, EPW // CH)
            def _(c):
                off = base + c * CH
                pltpu.sync_copy(j_hbm.at[pl.ds(off, CH)], jv)
                pltpu.sync_copy(pay_hbm.at[pl.ds(off, CH)], bufm)
                pltpu.sync_copy(bufm, accm.at[jv], add=True)

            plsc.subcore_barrier()
            pltpu.sync_copy(accm.at[pl.ds(sid * ZR, ZR)],
                            out_hbm.at[cid, pl.ds(sid * ZR, ZR)])
            plsc.subcore_barrier()

    return _sc_gather, _sc_scatter


def _unpack(t):
    hw = t[:, :HD]
    ha = lax.bitcast_convert_type(hw << 16, jnp.float32)
    hb = lax.bitcast_convert_type(hw & jnp.int32(-65536),
                                  jnp.float32)
    h = jnp.concatenate([ha, hb], axis=1).astype(jnp.bfloat16)
    x = lax.bitcast_convert_type(t[:, HD:HD + XW], jnp.float32)
    return h, x


def _edge_body(ti_ref, tj_ref, a_ref, we1_ref, be1_ref, we2_ref, be2_ref,
               wx1_ref, bx1_ref, wx2_ref, bx2_ref, winf_ref, binf_ref,
               outm_ref, outx_ref):
    hi, xi = _unpack(ti_ref[...])
    hj, xj = _unpack(tj_ref[...])
    diff = xi - xj
    dist = jnp.sqrt(jnp.sum(diff * diff, axis=1, keepdims=True) + 1e-12)

    extras = jnp.concatenate([dist, a_ref[...]], axis=1)
    hhda = jnp.concatenate(
        [hi, hj, extras.astype(jnp.bfloat16)], axis=1)

    e1 = jnp.dot(hhda, we1_ref[...],
                 preferred_element_type=jnp.float32) + be1_ref[...]
    e1 = e1 * jax.nn.sigmoid(e1)
    m1 = jnp.dot(e1.astype(jnp.bfloat16), we2_ref[...],
                 preferred_element_type=jnp.float32) + be2_ref[...]
    m_ij = m1 * jax.nn.sigmoid(m1)

    x1 = jnp.dot(hhda, wx1_ref[...],
                 preferred_element_type=jnp.float32) + bx1_ref[...]
    x1 = x1 * jax.nn.sigmoid(x1)
    w_x = jnp.sum(x1 * wx2_ref[...], axis=1, keepdims=True) + bx2_ref[...]

    gate = jax.nn.sigmoid(
        jnp.sum(m_ij * winf_ref[...], axis=1, keepdims=True) + binf_ref[...])

    cnt_col = (lax.broadcasted_iota(jnp.int32, (1, D), 1) == 3)
    diff_w = jnp.pad(diff * w_x, ((0, 0), (0, D - XW)))
    outm_ref[...] = gate * m_ij
    outx_ref[...] = diff_w + cnt_col.astype(jnp.float32)


BE = 2048


def _edge_call(ti, tj, a7, we1p, be1r, we2b, be2r, wx1p, bx1r, wx2r, bx2s,
               winfr, binfs):
    full = lambda shape: pl.BlockSpec(shape, lambda i: (0, 0))
    row = lambda width: pl.BlockSpec((BE, width), lambda i: (i, 0))
    return pl.pallas_call(
        _edge_body,
        grid=(E_PAD // BE,),
        in_specs=[
            row(D), row(D), row(7),
            full((264, D)), full((1, D)), full((D, D)), full((1, D)),
            full((264, D)), full((1, D)), full((1, D)), full((1, 1)),
            full((1, D)), full((1, 1)),
        ],
        out_specs=[row(D), row(D)],
        out_shape=[jax.ShapeDtypeStruct((E_PAD, D), jnp.float32),
                   jax.ShapeDtypeStruct((E_PAD, D), jnp.float32)],
    )(ti, tj, a7, we1p, be1r, we2b, be2r, wx1p, bx1r, wx2r, bx2s, winfr,
      binfs)


def _node_body(h_ref, xs_ref, m0_ref, m1_ref, p0_ref, p1_ref, wh1_ref,
               bh1_ref, wh2_ref, bh2_ref, hout_ref, xout_ref):
    dwc = p0_ref[:, :XW] + p1_ref[:, :XW]
    cnt_col = (lax.broadcasted_iota(jnp.int32, (1, XW), 1) == 3)
    cnt = jnp.sum(dwc * cnt_col.astype(jnp.float32), axis=1, keepdims=True)
    xout_ref[...] = xs_ref[...] + dwc / jnp.maximum(cnt, 1.0)

    h = h_ref[...]
    m = m0_ref[...] + m1_ref[...]
    hcat = jnp.concatenate([h, m], axis=1).astype(jnp.bfloat16)
    t = jnp.dot(hcat, wh1_ref[...],
                preferred_element_type=jnp.float32) + bh1_ref[...]
    t = t * jax.nn.sigmoid(t)
    hout_ref[...] = h + jnp.dot(t.astype(jnp.bfloat16), wh2_ref[...],
                                preferred_element_type=jnp.float32) + bh2_ref[...]


BN = 1000


def _node_call(h, xs, m0, m1, p0, p1, wh1b, bh1r, wh2b, bh2r):
    full = lambda shape: pl.BlockSpec(shape, lambda i: (0, 0))
    row = lambda width: pl.BlockSpec((BN, width), lambda i: (i, 0))
    return pl.pallas_call(
        _node_body,
        grid=(N_NODES // BN,),
        in_specs=[
            row(D), row(XW), row(D), row(D), row(D), row(D),
            full((2 * D, D)), full((1, D)), full((D, D)), full((1, D)),
        ],
        out_specs=[row(D), row(XW)],
        out_shape=[jax.ShapeDtypeStruct((N_NODES, D), jnp.float32),
                   jax.ShapeDtypeStruct((N_NODES, XW), jnp.float32)],
    )(h, xs, m0, m1, p0, p1, wh1b, bh1r, wh2b, bh2r)


def _pack_table(h, x):
    hb = h.astype(jnp.bfloat16)
    lo = lax.bitcast_convert_type(hb[:, :HD], jnp.uint16).astype(jnp.uint32)
    hi = lax.bitcast_convert_type(hb[:, HD:], jnp.uint16).astype(jnp.uint32)
    hw = lax.bitcast_convert_type(lo | (hi << 16), jnp.int32)
    xw = lax.bitcast_convert_type(
        jnp.pad(x, ((0, 0), (0, HD - 3))), jnp.int32)
    return jnp.concatenate([hw, xw[:, :D - HD]], axis=1)


def kernel(h, x, edge_index, a_ij, We1, be1, We2, be2, Wx1, bx1, Wx2, bx2,
           Wh1, bh1, Wh2, bh2, Winf, binf):
    f32 = jnp.float32
    bf16 = jnp.bfloat16

    tbl = _pack_table(h, x)

    pad_e = E_PAD - E_EDGES
    spread = (jnp.arange(pad_e, dtype=jnp.int32) * 37) % N_NODES
    i_idx = jnp.concatenate([edge_index[0], spread])
    jg_idx = jnp.concatenate([edge_index[1], spread])
    js_idx = jnp.pad(edge_index[1], (0, pad_e),
                     constant_values=N_NODES)
    a7 = jnp.pad(a_ij, ((0, pad_e), (0, 3)))

    pack = lambda W: jnp.concatenate([W, jnp.zeros((3, D), f32)], axis=0)
    we1p = pack(We1).astype(bf16)
    wx1p = pack(Wx1).astype(bf16)
    we2b = We2.astype(bf16)
    wh1b = Wh1.astype(bf16)
    wh2b = Wh2.astype(bf16)
    r = lambda v: v.reshape(1, -1)

    sc_gather, sc_scatter = _sc_kernels()
    ti, tj = sc_gather(tbl, i_idx, jg_idx)
    paym, payx = _edge_call(ti, tj, a7, we1p, r(be1), we2b, r(be2), wx1p,
                            r(bx1), r(Wx2), r(bx2), r(Winf), r(binf))
    zm = jnp.zeros((ZR, D), f32)
    pm, px = sc_scatter(paym, payx, js_idx, zm)
    xs = jnp.pad(x, ((0, 0), (0, XW - 3)))
    h_new, x8 = _node_call(h, xs, pm[0, :N_NODES], pm[1, :N_NODES],
                           px[0, :N_NODES], px[1, :N_NODES],
                           wh1b, r(bh1), wh2b, r(bh2))
    return (h_new, x8[:, :3])


# speedup vs baseline: 3.8243x; 1.1223x over previous
import functools

import jax
import jax.numpy as jnp
from jax import lax
from jax.experimental import pallas as pl
from jax.experimental.pallas import tpu as pltpu
from jax.experimental.pallas import tpu_sc as plsc

N_NODES = 10000
E_EDGES = 320000
D = 128
HD = D // 2
NC, NS = 2, 16
NW = NC * NS
CH = 128
E_PAD = 327680
NCK = 2
EH = E_PAD // NCK
EPW = EH // NW
XW = 8
NROW = 10240
ZR = NROW // NS


@functools.lru_cache(maxsize=None)
def _sc_kernels():
    mesh = plsc.VectorSubcoreMesh(core_axis_name="c", subcore_axis_name="s")

    @functools.partial(
        pl.kernel,
        mesh=mesh,
        out_type=(jax.ShapeDtypeStruct((EH, D), jnp.int32),
                  jax.ShapeDtypeStruct((EH, D), jnp.int32)),
        scratch_types=[
            pltpu.VMEM((EPW,), jnp.int32),
            pltpu.VMEM((EPW,), jnp.int32),
            pltpu.VMEM((CH, D), jnp.int32),
            pltpu.VMEM((CH, D), jnp.int32),
            pltpu.VMEM((CH, D), jnp.int32),
            pltpu.VMEM((CH, D), jnp.int32),
            pltpu.SemaphoreType.DMA,
            pltpu.SemaphoreType.DMA,
            pltpu.SemaphoreType.DMA,
            pltpu.SemaphoreType.DMA,
        ],
    )
    def _sc_gather(tbl_hbm, i_hbm, j_hbm, ti_hbm, tj_hbm,
                   iva, jva, bi0, bj0, bi1, bj1, sg0, sg1, sw0, sw1):
        base = (lax.axis_index("c") * (EH // NC)
                + lax.axis_index("s") * EPW)
        G = EPW // CH // 2

        pltpu.sync_copy(i_hbm.at[pl.ds(base, EPW)], iva)
        pltpu.sync_copy(j_hbm.at[pl.ds(base, EPW)], jva)

        ixi = lambda c: tbl_hbm.at[iva.at[pl.ds(c * CH, CH)]]
        ixj = lambda c: tbl_hbm.at[jva.at[pl.ds(c * CH, CH)]]
        out_i = lambda c: ti_hbm.at[pl.ds(base + c * CH, CH)]
        out_j = lambda c: tj_hbm.at[pl.ds(base + c * CH, CH)]

        pltpu.async_copy(ixi(0), bi0, sg0)
        pltpu.async_copy(ixj(0), bj0, sg0)

        @pl.loop(0, G)
        def _(g):
            c0 = 2 * g
            c1 = c0 + 1
            @pl.when(g > 0)
            def _():
                pltpu.make_async_copy(bi1, out_i(c1), sw1).wait()
                pltpu.make_async_copy(bj1, out_j(c1), sw1).wait()
            pltpu.async_copy(ixi(c1), bi1, sg1)
            pltpu.async_copy(ixj(c1), bj1, sg1)
            pltpu.make_async_copy(ixi(c0), bi0, sg0).wait()
            pltpu.make_async_copy(ixj(c0), bj0, sg0).wait()
            pltpu.async_copy(bi0, out_i(c0), sw0)
            pltpu.async_copy(bj0, out_j(c0), sw0)
            pltpu.make_async_copy(bi0, out_i(c0), sw0).wait()
            pltpu.make_async_copy(bj0, out_j(c0), sw0).wait()

            @pl.when(g < G - 1)
            def _():
                pltpu.async_copy(ixi(c0 + 2), bi0, sg0)
                pltpu.async_copy(ixj(c0 + 2), bj0, sg0)
            pltpu.make_async_copy(ixi(c1), bi1, sg1).wait()
            pltpu.make_async_copy(ixj(c1), bj1, sg1).wait()
            pltpu.async_copy(bi1, out_i(c1), sw1)
            pltpu.async_copy(bj1, out_j(c1), sw1)

        pltpu.make_async_copy(bi1, out_i(1), sw1).wait()
        pltpu.make_async_copy(bj1, out_j(1), sw1).wait()

    @functools.partial(
        pl.kernel,
        mesh=mesh,
        out_type=(jax.ShapeDtypeStruct((NC, NROW, D), jnp.float32),
                  jax.ShapeDtypeStruct((NC, NROW, D), jnp.float32)),
        scratch_types=[
            pltpu.VMEM((CH,), jnp.int32),
            pltpu.VMEM((CH, D), jnp.float32),
            pltpu.VMEM_SHARED((NROW, D), jnp.float32),
        ],
    )
    def _sc_scatter(paym_hbm, payx_hbm, j_hbm, zm_hbm, outm_hbm, outx_hbm,
                    jv, bufm, accm):
        cid = lax.axis_index("c")
        sid = lax.axis_index("s")
        base = cid * (EH // NC) + sid * EPW

        for pay_hbm, out_hbm in ((paym_hbm, outm_hbm), (payx_hbm, outx_hbm)):
            pltpu.sync_copy(zm_hbm, accm.at[pl.ds(sid * ZR, ZR)])
            plsc.subcore_barrier()

            @pl.loop(0, EPW // CH)
            def _(c):
                off = base + c * CH
                pltpu.sync_copy(j_hbm.at[pl.ds(off, CH)], jv)
                pltpu.sync_copy(pay_hbm.at[pl.ds(off, CH)], bufm)
                pltpu.sync_copy(bufm, accm.at[jv], add=True)

            plsc.subcore_barrier()
            pltpu.sync_copy(accm.at[pl.ds(sid * ZR, ZR)],
                            out_hbm.at[cid, pl.ds(sid * ZR, ZR)])
            plsc.subcore_barrier()

    return _sc_gather, _sc_scatter


def _unpack(t):
    hw = t[:, :HD]
    ha = lax.bitcast_convert_type(hw << 16, jnp.float32)
    hb = lax.bitcast_convert_type(hw & jnp.int32(-65536),
                                  jnp.float32)
    h = jnp.concatenate([ha, hb], axis=1).astype(jnp.bfloat16)
    x = lax.bitcast_convert_type(t[:, HD:HD + XW], jnp.float32)
    return h, x


def _edge_body(ti_ref, tj_ref, a_ref, we1_ref, be1_ref, we2_ref, be2_ref,
               wx1_ref, bx1_ref, wx2_ref, bx2_ref, winf_ref, binf_ref,
               outm_ref, outx_ref):
    hi, xi = _unpack(ti_ref[...])
    hj, xj = _unpack(tj_ref[...])
    diff = xi - xj
    dist = jnp.sqrt(jnp.sum(diff * diff, axis=1, keepdims=True) + 1e-12)

    extras = jnp.concatenate([dist, a_ref[...]], axis=1)
    hhda = jnp.concatenate(
        [hi, hj, extras.astype(jnp.bfloat16)], axis=1)

    e1 = jnp.dot(hhda, we1_ref[...],
                 preferred_element_type=jnp.float32) + be1_ref[...]
    e1 = e1 * jax.nn.sigmoid(e1)
    m1 = jnp.dot(e1.astype(jnp.bfloat16), we2_ref[...],
                 preferred_element_type=jnp.float32) + be2_ref[...]
    m_ij = m1 * jax.nn.sigmoid(m1)

    x1 = jnp.dot(hhda, wx1_ref[...],
                 preferred_element_type=jnp.float32) + bx1_ref[...]
    x1 = x1 * jax.nn.sigmoid(x1)
    w_x = jnp.sum(x1 * wx2_ref[...], axis=1, keepdims=True) + bx2_ref[...]

    gate = jax.nn.sigmoid(
        jnp.sum(m_ij * winf_ref[...], axis=1, keepdims=True) + binf_ref[...])

    cnt_col = (lax.broadcasted_iota(jnp.int32, (1, D), 1) == 3)
    diff_w = jnp.pad(diff * w_x, ((0, 0), (0, D - XW)))
    outm_ref[...] = gate * m_ij
    outx_ref[...] = diff_w + cnt_col.astype(jnp.float32)


BE = 2048


def _edge_call(ti, tj, a7, we1p, be1r, we2b, be2r, wx1p, bx1r, wx2r, bx2s,
               winfr, binfs):
    full = lambda shape: pl.BlockSpec(shape, lambda i: (0, 0))
    row = lambda width: pl.BlockSpec((BE, width), lambda i: (i, 0))
    return pl.pallas_call(
        _edge_body,
        grid=(EH // BE,),
        in_specs=[
            row(D), row(D), row(7),
            full((264, D)), full((1, D)), full((D, D)), full((1, D)),
            full((264, D)), full((1, D)), full((1, D)), full((1, 1)),
            full((1, D)), full((1, 1)),
        ],
        out_specs=[row(D), row(D)],
        out_shape=[jax.ShapeDtypeStruct((EH, D), jnp.float32),
                   jax.ShapeDtypeStruct((EH, D), jnp.float32)],
    )(ti, tj, a7, we1p, be1r, we2b, be2r, wx1p, bx1r, wx2r, bx2s, winfr,
      binfs)


def _node_body(h_ref, xs_ref, m0_ref, m1_ref, m2_ref, m3_ref,
               p0_ref, p1_ref, p2_ref, p3_ref, wh1_ref,
               bh1_ref, wh2_ref, bh2_ref, hout_ref, xout_ref):
    dwc = (p0_ref[:, :XW] + p1_ref[:, :XW]
           + p2_ref[:, :XW] + p3_ref[:, :XW])
    cnt_col = (lax.broadcasted_iota(jnp.int32, (1, XW), 1) == 3)
    cnt = jnp.sum(dwc * cnt_col.astype(jnp.float32), axis=1, keepdims=True)
    xout_ref[...] = xs_ref[...] + dwc / jnp.maximum(cnt, 1.0)

    h = h_ref[...]
    m = m0_ref[...] + m1_ref[...] + m2_ref[...] + m3_ref[...]
    hcat = jnp.concatenate([h, m], axis=1).astype(jnp.bfloat16)
    t = jnp.dot(hcat, wh1_ref[...],
                preferred_element_type=jnp.float32) + bh1_ref[...]
    t = t * jax.nn.sigmoid(t)
    hout_ref[...] = h + jnp.dot(t.astype(jnp.bfloat16), wh2_ref[...],
                                preferred_element_type=jnp.float32) + bh2_ref[...]


BN = 1000


def _node_call(h, xs, ms, ps, wh1b, bh1r, wh2b, bh2r):
    full = lambda shape: pl.BlockSpec(shape, lambda i: (0, 0))
    row = lambda width: pl.BlockSpec((BN, width), lambda i: (i, 0))
    return pl.pallas_call(
        _node_body,
        grid=(N_NODES // BN,),
        in_specs=[
            row(D), row(XW), row(D), row(D), row(D), row(D),
            row(D), row(D), row(D), row(D),
            full((2 * D, D)), full((1, D)), full((D, D)), full((1, D)),
        ],
        out_specs=[row(D), row(XW)],
        out_shape=[jax.ShapeDtypeStruct((N_NODES, D), jnp.float32),
                   jax.ShapeDtypeStruct((N_NODES, XW), jnp.float32)],
    )(h, xs, *ms, *ps, wh1b, bh1r, wh2b, bh2r)


def _pack_table(h, x):
    hb = h.astype(jnp.bfloat16)
    lo = lax.bitcast_convert_type(hb[:, :HD], jnp.uint16).astype(jnp.uint32)
    hi = lax.bitcast_convert_type(hb[:, HD:], jnp.uint16).astype(jnp.uint32)
    hw = lax.bitcast_convert_type(lo | (hi << 16), jnp.int32)
    xw = lax.bitcast_convert_type(
        jnp.pad(x, ((0, 0), (0, HD - 3))), jnp.int32)
    return jnp.concatenate([hw, xw[:, :D - HD]], axis=1)


def kernel(h, x, edge_index, a_ij, We1, be1, We2, be2, Wx1, bx1, Wx2, bx2,
           Wh1, bh1, Wh2, bh2, Winf, binf):
    f32 = jnp.float32
    bf16 = jnp.bfloat16

    tbl = _pack_table(h, x)

    pad_e = E_PAD - E_EDGES
    spread = (jnp.arange(pad_e, dtype=jnp.int32) * 37) % N_NODES
    i_idx = jnp.concatenate([edge_index[0], spread])
    jg_idx = jnp.concatenate([edge_index[1], spread])
    js_idx = jnp.pad(edge_index[1], (0, pad_e),
                     constant_values=N_NODES)
    a7 = jnp.pad(a_ij, ((0, pad_e), (0, 3)))

    pack = lambda W: jnp.concatenate([W, jnp.zeros((3, D), f32)], axis=0)
    we1p = pack(We1).astype(bf16)
    wx1p = pack(Wx1).astype(bf16)
    we2b = We2.astype(bf16)
    wh1b = Wh1.astype(bf16)
    wh2b = Wh2.astype(bf16)
    r = lambda v: v.reshape(1, -1)

    sc_gather, sc_scatter = _sc_kernels()
    zm = jnp.zeros((ZR, D), f32)
    ms, ps = [], []
    gathered = [sc_gather(tbl, i_idx[k * EH:(k + 1) * EH],
                          jg_idx[k * EH:(k + 1) * EH]) for k in range(NCK)]
    for k in range(NCK):
        ti, tj = gathered[k]
        paym, payx = _edge_call(ti, tj, a7[k * EH:(k + 1) * EH], we1p,
                                r(be1), we2b, r(be2), wx1p, r(bx1), r(Wx2),
                                r(bx2), r(Winf), r(binf))
        pm, px = sc_scatter(paym, payx, js_idx[k * EH:(k + 1) * EH], zm)
        ms += [pm[0, :N_NODES], pm[1, :N_NODES]]
        ps += [px[0, :N_NODES], px[1, :N_NODES]]
    xs = jnp.pad(x, ((0, 0), (0, XW - 3)))
    h_new, x8 = _node_call(h, xs, ms, ps, wh1b, r(bh1), wh2b, r(bh2))
    return (h_new, x8[:, :3])


# speedup vs baseline: 4.0189x; 1.0509x over previous
import functools

import jax
import jax.numpy as jnp
from jax import lax
from jax.experimental import pallas as pl
from jax.experimental.pallas import tpu as pltpu
from jax.experimental.pallas import tpu_sc as plsc

N_NODES = 10000
E_EDGES = 320000
D = 128
HD = D // 2
NC, NS = 2, 16
NW = NC * NS
CH = 128
E_PAD = 327680
NCK = 4
EH = E_PAD // NCK
EPW = EH // NW
XW = 8
NROW = 10240
ZR = NROW // NS


@functools.lru_cache(maxsize=None)
def _sc_kernels():
    mesh = plsc.VectorSubcoreMesh(core_axis_name="c", subcore_axis_name="s")

    @functools.partial(
        pl.kernel,
        mesh=mesh,
        out_type=(jax.ShapeDtypeStruct((EH, D), jnp.int32),
                  jax.ShapeDtypeStruct((EH, D), jnp.int32)),
        scratch_types=[
            pltpu.VMEM((EPW,), jnp.int32),
            pltpu.VMEM((EPW,), jnp.int32),
            pltpu.VMEM((CH, D), jnp.int32),
            pltpu.VMEM((CH, D), jnp.int32),
            pltpu.VMEM((CH, D), jnp.int32),
            pltpu.VMEM((CH, D), jnp.int32),
            pltpu.SemaphoreType.DMA,
            pltpu.SemaphoreType.DMA,
            pltpu.SemaphoreType.DMA,
            pltpu.SemaphoreType.DMA,
        ],
    )
    def _sc_gather(tbl_hbm, i_hbm, j_hbm, ti_hbm, tj_hbm,
                   iva, jva, bi0, bj0, bi1, bj1, sg0, sg1, sw0, sw1):
        base = (lax.axis_index("c") * (EH // NC)
                + lax.axis_index("s") * EPW)
        G = EPW // CH // 2

        pltpu.sync_copy(i_hbm.at[pl.ds(base, EPW)], iva)
        pltpu.sync_copy(j_hbm.at[pl.ds(base, EPW)], jva)

        ixi = lambda c: tbl_hbm.at[iva.at[pl.ds(c * CH, CH)]]
        ixj = lambda c: tbl_hbm.at[jva.at[pl.ds(c * CH, CH)]]
        out_i = lambda c: ti_hbm.at[pl.ds(base + c * CH, CH)]
        out_j = lambda c: tj_hbm.at[pl.ds(base + c * CH, CH)]

        pltpu.async_copy(ixi(0), bi0, sg0)
        pltpu.async_copy(ixj(0), bj0, sg0)

        @pl.loop(0, G)
        def _(g):
            c0 = 2 * g
            c1 = c0 + 1
            @pl.when(g > 0)
            def _():
                pltpu.make_async_copy(bi1, out_i(c1), sw1).wait()
                pltpu.make_async_copy(bj1, out_j(c1), sw1).wait()
            pltpu.async_copy(ixi(c1), bi1, sg1)
            pltpu.async_copy(ixj(c1), bj1, sg1)
            pltpu.make_async_copy(ixi(c0), bi0, sg0).wait()
            pltpu.make_async_copy(ixj(c0), bj0, sg0).wait()
            pltpu.async_copy(bi0, out_i(c0), sw0)
            pltpu.async_copy(bj0, out_j(c0), sw0)
            pltpu.make_async_copy(bi0, out_i(c0), sw0).wait()
            pltpu.make_async_copy(bj0, out_j(c0), sw0).wait()

            @pl.when(g < G - 1)
            def _():
                pltpu.async_copy(ixi(c0 + 2), bi0, sg0)
                pltpu.async_copy(ixj(c0 + 2), bj0, sg0)
            pltpu.make_async_copy(ixi(c1), bi1, sg1).wait()
            pltpu.make_async_copy(ixj(c1), bj1, sg1).wait()
            pltpu.async_copy(bi1, out_i(c1), sw1)
            pltpu.async_copy(bj1, out_j(c1), sw1)

        pltpu.make_async_copy(bi1, out_i(1), sw1).wait()
        pltpu.make_async_copy(bj1, out_j(1), sw1).wait()

    @functools.partial(
        pl.kernel,
        mesh=mesh,
        out_type=(jax.ShapeDtypeStruct((NC, NROW, D), jnp.float32),
                  jax.ShapeDtypeStruct((NC, NROW, D), jnp.float32)),
        scratch_types=[
            pltpu.VMEM((CH,), jnp.int32),
            pltpu.VMEM((CH, D), jnp.float32),
            pltpu.VMEM_SHARED((NROW, D), jnp.float32),
        ],
    )
    def _sc_scatter(paym_hbm, payx_hbm, j_hbm, zm_hbm, outm_hbm, outx_hbm,
                    jv, bufm, accm):
        cid = lax.axis_index("c")
        sid = lax.axis_index("s")
        base = cid * (EH // NC) + sid * EPW

        for pay_hbm, out_hbm in ((paym_hbm, outm_hbm), (payx_hbm, outx_hbm)):
            pltpu.sync_copy(zm_hbm, accm.at[pl.ds(sid * ZR, ZR)])
            plsc.subcore_barrier()

            @pl.loop(0, EPW // CH)
            def _(c):
                off = base + c * CH
                pltpu.sync_copy(j_hbm.at[pl.ds(off, CH)], jv)
                pltpu.sync_copy(pay_hbm.at[pl.ds(off, CH)], bufm)
                pltpu.sync_copy(bufm, accm.at[jv], add=True)

            plsc.subcore_barrier()
            pltpu.sync_copy(accm.at[pl.ds(sid * ZR, ZR)],
                            out_hbm.at[cid, pl.ds(sid * ZR, ZR)])
            plsc.subcore_barrier()

    return _sc_gather, _sc_scatter


def _unpack(t):
    hw = t[:, :HD]
    ha = lax.bitcast_convert_type(hw << 16, jnp.float32)
    hb = lax.bitcast_convert_type(hw & jnp.int32(-65536),
                                  jnp.float32)
    h = jnp.concatenate([ha, hb], axis=1).astype(jnp.bfloat16)
    x = lax.bitcast_convert_type(t[:, HD:HD + XW], jnp.float32)
    return h, x


def _edge_body(ti_ref, tj_ref, a_ref, we1_ref, be1_ref, we2_ref, be2_ref,
               wx1_ref, bx1_ref, wx2_ref, bx2_ref, winf_ref, binf_ref,
               outm_ref, outx_ref):
    hi, xi = _unpack(ti_ref[...])
    hj, xj = _unpack(tj_ref[...])
    diff = xi - xj
    dist = jnp.sqrt(jnp.sum(diff * diff, axis=1, keepdims=True) + 1e-12)

    extras = jnp.concatenate([dist, a_ref[...]], axis=1)
    hhda = jnp.concatenate(
        [hi, hj, extras.astype(jnp.bfloat16)], axis=1)

    e1 = jnp.dot(hhda, we1_ref[...],
                 preferred_element_type=jnp.float32) + be1_ref[...]
    e1 = e1 * jax.nn.sigmoid(e1)
    m1 = jnp.dot(e1.astype(jnp.bfloat16), we2_ref[...],
                 preferred_element_type=jnp.float32) + be2_ref[...]
    m_ij = m1 * jax.nn.sigmoid(m1)

    x1 = jnp.dot(hhda, wx1_ref[...],
                 preferred_element_type=jnp.float32) + bx1_ref[...]
    x1 = x1 * jax.nn.sigmoid(x1)
    w_x = jnp.sum(x1 * wx2_ref[...], axis=1, keepdims=True) + bx2_ref[...]

    gate = jax.nn.sigmoid(
        jnp.sum(m_ij * winf_ref[...], axis=1, keepdims=True) + binf_ref[...])

    cnt_col = (lax.broadcasted_iota(jnp.int32, (1, D), 1) == 3)
    diff_w = jnp.pad(diff * w_x, ((0, 0), (0, D - XW)))
    outm_ref[...] = gate * m_ij
    outx_ref[...] = diff_w + cnt_col.astype(jnp.float32)


BE = 2048


def _edge_call(ti, tj, a7, we1p, be1r, we2b, be2r, wx1p, bx1r, wx2r, bx2s,
               winfr, binfs):
    full = lambda shape: pl.BlockSpec(shape, lambda i: (0, 0))
    row = lambda width: pl.BlockSpec((BE, width), lambda i: (i, 0))
    return pl.pallas_call(
        _edge_body,
        grid=(EH // BE,),
        in_specs=[
            row(D), row(D), row(7),
            full((264, D)), full((1, D)), full((D, D)), full((1, D)),
            full((264, D)), full((1, D)), full((1, D)), full((1, 1)),
            full((1, D)), full((1, 1)),
        ],
        out_specs=[row(D), row(D)],
        out_shape=[jax.ShapeDtypeStruct((EH, D), jnp.float32),
                   jax.ShapeDtypeStruct((EH, D), jnp.float32)],
    )(ti, tj, a7, we1p, be1r, we2b, be2r, wx1p, bx1r, wx2r, bx2s, winfr,
      binfs)


def _node_body(h_ref, xs_ref, *rest):
    m_refs = rest[:2 * NCK]
    p_refs = rest[2 * NCK:4 * NCK]
    (wh1_ref, bh1_ref, wh2_ref, bh2_ref, hout_ref, xout_ref) = rest[4 * NCK:]
    dwc = p_refs[0][:, :XW]
    for p in p_refs[1:]:
        dwc = dwc + p[:, :XW]
    cnt_col = (lax.broadcasted_iota(jnp.int32, (1, XW), 1) == 3)
    cnt = jnp.sum(dwc * cnt_col.astype(jnp.float32), axis=1, keepdims=True)
    xout_ref[...] = xs_ref[...] + dwc / jnp.maximum(cnt, 1.0)

    h = h_ref[...]
    m = m_refs[0][...]
    for mr in m_refs[1:]:
        m = m + mr[...]
    hcat = jnp.concatenate([h, m], axis=1).astype(jnp.bfloat16)
    t = jnp.dot(hcat, wh1_ref[...],
                preferred_element_type=jnp.float32) + bh1_ref[...]
    t = t * jax.nn.sigmoid(t)
    hout_ref[...] = h + jnp.dot(t.astype(jnp.bfloat16), wh2_ref[...],
                                preferred_element_type=jnp.float32) + bh2_ref[...]


BN = 1000


def _node_call(h, xs, ms, ps, wh1b, bh1r, wh2b, bh2r):
    full = lambda shape: pl.BlockSpec(shape, lambda i: (0, 0))
    row = lambda width: pl.BlockSpec((BN, width), lambda i: (i, 0))
    return pl.pallas_call(
        _node_body,
        grid=(N_NODES // BN,),
        in_specs=(
            [row(D), row(XW)] + [row(D)] * (4 * NCK)
            + [full((2 * D, D)), full((1, D)), full((D, D)), full((1, D))]
        ),
        out_specs=[row(D), row(XW)],
        out_shape=[jax.ShapeDtypeStruct((N_NODES, D), jnp.float32),
                   jax.ShapeDtypeStruct((N_NODES, XW), jnp.float32)],
    )(h, xs, *ms, *ps, wh1b, bh1r, wh2b, bh2r)


def _pack_table(h, x):
    hb = h.astype(jnp.bfloat16)
    lo = lax.bitcast_convert_type(hb[:, :HD], jnp.uint16).astype(jnp.uint32)
    hi = lax.bitcast_convert_type(hb[:, HD:], jnp.uint16).astype(jnp.uint32)
    hw = lax.bitcast_convert_type(lo | (hi << 16), jnp.int32)
    xw = lax.bitcast_convert_type(
        jnp.pad(x, ((0, 0), (0, HD - 3))), jnp.int32)
    return jnp.concatenate([hw, xw[:, :D - HD]], axis=1)


def kernel(h, x, edge_index, a_ij, We1, be1, We2, be2, Wx1, bx1, Wx2, bx2,
           Wh1, bh1, Wh2, bh2, Winf, binf):
    f32 = jnp.float32
    bf16 = jnp.bfloat16

    tbl = _pack_table(h, x)

    pad_e = E_PAD - E_EDGES
    spread = (jnp.arange(pad_e, dtype=jnp.int32) * 37) % N_NODES
    i_idx = jnp.concatenate([edge_index[0], spread])
    jg_idx = jnp.concatenate([edge_index[1], spread])
    js_idx = jnp.pad(edge_index[1], (0, pad_e),
                     constant_values=N_NODES)
    a7 = jnp.pad(a_ij, ((0, pad_e), (0, 3)))

    pack = lambda W: jnp.concatenate([W, jnp.zeros((3, D), f32)], axis=0)
    we1p = pack(We1).astype(bf16)
    wx1p = pack(Wx1).astype(bf16)
    we2b = We2.astype(bf16)
    wh1b = Wh1.astype(bf16)
    wh2b = Wh2.astype(bf16)
    r = lambda v: v.reshape(1, -1)

    sc_gather, sc_scatter = _sc_kernels()
    zm = jnp.zeros((ZR, D), f32)
    ms, ps = [], []
    for k in range(NCK):
        ti, tj = sc_gather(tbl, i_idx[k * EH:(k + 1) * EH],
                           jg_idx[k * EH:(k + 1) * EH])
        paym, payx = _edge_call(ti, tj, a7[k * EH:(k + 1) * EH], we1p,
                                r(be1), we2b, r(be2), wx1p, r(bx1), r(Wx2),
                                r(bx2), r(Winf), r(binf))
        pm, px = sc_scatter(paym, payx, js_idx[k * EH:(k + 1) * EH], zm)
        ms += [pm[0, :N_NODES], pm[1, :N_NODES]]
        ps += [px[0, :N_NODES], px[1, :N_NODES]]
    xs = jnp.pad(x, ((0, 0), (0, XW - 3)))
    h_new, x8 = _node_call(h, xs, ms, ps, wh1b, r(bh1), wh2b, r(bh2))
    return (h_new, x8[:, :3])
